# baseline (speedup 1.0000x reference)
import jax
import jax.numpy as jnp
from jax.experimental import pallas as pl

_BT = 1024


def _matmul_kernel(x_ref, wt_ref, o_ref):
    o_ref[...] = jax.lax.dot_general(
        x_ref[...],
        wt_ref[...],
        dimension_numbers=(((1,), (0,)), ((), ())),
        precision=jax.lax.Precision.DEFAULT,
        preferred_element_type=jnp.float32,
    )


def kernel(x, W):
    T, d_model = x.shape
    n_experts = W.shape[0]
    wt = W.T
    grid = (T // _BT,)
    return pl.pallas_call(
        _matmul_kernel,
        grid=grid,
        in_specs=[
            pl.BlockSpec((_BT, d_model), lambda i: (i, 0)),
            pl.BlockSpec((d_model, n_experts), lambda i: (0, 0)),
        ],
        out_specs=pl.BlockSpec((_BT, n_experts), lambda i: (i, 0)),
        out_shape=jax.ShapeDtypeStruct((T, n_experts), jnp.float32),
    )(x, wt)


# speedup vs baseline: 1.0190x; 1.0190x over previous
import jax
import jax.numpy as jnp
from jax.experimental import pallas as pl

_BT = 1024


def _matmul_kernel(x_ref, w_ref, o_ref):
    o_ref[...] = jax.lax.dot_general(
        x_ref[...],
        w_ref[...],
        dimension_numbers=(((1,), (1,)), ((), ())),
        precision=jax.lax.Precision.DEFAULT,
        preferred_element_type=jnp.float32,
    )


def kernel(x, W):
    T, d_model = x.shape
    n_experts = W.shape[0]
    grid = (T // _BT,)
    return pl.pallas_call(
        _matmul_kernel,
        grid=grid,
        in_specs=[
            pl.BlockSpec((_BT, d_model), lambda i: (i, 0)),
            pl.BlockSpec((n_experts, d_model), lambda i: (0, 0)),
        ],
        out_specs=pl.BlockSpec((_BT, n_experts), lambda i: (i, 0)),
        out_shape=jax.ShapeDtypeStruct((T, n_experts), jnp.float32),
    )(x, W)
